# baseline (speedup 1.0000x reference)
import functools

import jax
import jax.numpy as jnp
from jax import lax
from jax.experimental import pallas as pl
from jax.experimental.pallas import tpu as pltpu
from jax.experimental.pallas import tpu_sc as plsc

DIM = 32
N = 10000
E = 160000
G = 500

NC, NS = 2, 16
NW = NC * NS
CH = 128
NCHUNK = 40
EW = CH * NCHUNK
E2 = EW * NW
REAL_CHUNKS_LAST = (E - (NW - 1) * EW) // CH

EBLK = 1024
NBLK = 2000
N2 = 10240
SROWS = N2 // NS


def _lin0_body(x_ref, w_ref, b_ref, o_ref):
    o_ref[...] = jnp.maximum(
        jnp.dot(x_ref[...], w_ref[...], preferred_element_type=jnp.float32,
                precision=lax.Precision.HIGHEST)
        + b_ref[...], 0.0)


def _edge_body(ea_ref, xj_ref, w1_ref, b1_ref, w2_ref, b2_ref, o_ref):
    eh = jnp.maximum(
        jnp.dot(ea_ref[...], w1_ref[...], preferred_element_type=jnp.float32,
                precision=lax.Precision.HIGHEST)
        + b1_ref[...], 0.0)
    theta = jnp.dot(eh, w2_ref[...], preferred_element_type=jnp.float32,
                    precision=lax.Precision.HIGHEST) + b2_ref[...]
    xj = xj_ref[:, 0:DIM]
    acc = xj[:, 0:1] * theta[:, 0:DIM]
    for i in range(1, DIM):
        acc = acc + xj[:, i:i + 1] * theta[:, i * DIM:(i + 1) * DIM]
    row = pl.program_id(0) * EBLK + lax.broadcasted_iota(jnp.int32, (EBLK, 1), 0)
    o_ref[...] = jnp.where(row < E, acc, 0.0)


def _node_body(aggp_ref, degp_ref, out_ref, cr_ref, cb_ref, wih_ref, whh_ref,
               bih_ref, bhh_ref, o_ref):
    agg = aggp_ref[0] + aggp_ref[1]
    deg = jnp.maximum(degp_ref[0] + degp_ref[1], 1.0)
    h = out_ref[:, 0:DIM]
    m = jnp.maximum(
        agg / deg
        + jnp.dot(h, cr_ref[...], preferred_element_type=jnp.float32,
                  precision=lax.Precision.HIGHEST)
        + cb_ref[...], 0.0)
    gi = jnp.dot(m, wih_ref[...], preferred_element_type=jnp.float32,
                 precision=lax.Precision.HIGHEST) + bih_ref[...]
    gh = jnp.dot(h, whh_ref[...], preferred_element_type=jnp.float32,
                 precision=lax.Precision.HIGHEST) + bhh_ref[...]
    r = jax.nn.sigmoid(gi[:, 0:DIM] + gh[:, 0:DIM])
    z = jax.nn.sigmoid(gi[:, DIM:2 * DIM] + gh[:, DIM:2 * DIM])
    n = jnp.tanh(gi[:, 2 * DIM:] + r * gh[:, 2 * DIM:])
    o_ref[...] = jnp.concatenate(
        [(1.0 - z) * n + z * h, jnp.zeros((NBLK, 128 - DIM), jnp.float32)], axis=1)


def _s2s_body(out_ref, b_ref, wih_ref, whh_ref, bih_ref, bhh_ref,
              l1_ref, l1b_ref, l2_ref, l2b_ref, o_ref, e_ref):
    nblk = 10
    chn = N // nblk
    gids = lax.broadcasted_iota(jnp.int32, (1, G), 1)
    dn = (((0,), (0,)), ((), ()))
    qstar = jnp.zeros((G, 2 * DIM), jnp.float32)
    hh = jnp.zeros((G, DIM), jnp.float32)
    cc = jnp.zeros((G, DIM), jnp.float32)
    for _ in range(3):
        g = (jnp.dot(qstar, wih_ref[...], preferred_element_type=jnp.float32,
                     precision=lax.Precision.HIGHEST)
             + bih_ref[...]
             + jnp.dot(hh, whh_ref[...], preferred_element_type=jnp.float32,
                       precision=lax.Precision.HIGHEST)
             + bhh_ref[...])
        ii = jax.nn.sigmoid(g[:, 0:DIM])
        ff = jax.nn.sigmoid(g[:, DIM:2 * DIM])
        gg = jnp.tanh(g[:, 2 * DIM:3 * DIM])
        oo = jax.nn.sigmoid(g[:, 3 * DIM:])
        cc = ff * cc + ii * gg
        hh = oo * jnp.tanh(cc)
        q = hh

        def pass1(bk, emax):
            st = pl.multiple_of(bk * chn, chn)
            bb = b_ref[pl.ds(st, chn), :]
            mff = (bb == gids).astype(jnp.float32)
            ob = out_ref[pl.ds(st, chn), 0:DIM]
            qg = jnp.dot(mff, q, preferred_element_type=jnp.float32,
                         precision=lax.Precision.HIGHEST)
            e = jnp.sum(ob * qg, axis=1, keepdims=True)
            e_ref[pl.ds(st, chn), :] = e
            return jnp.maximum(
                emax,
                jnp.max(mff * e + (mff - 1.0) * 1e30, axis=0, keepdims=True))

        emax = lax.fori_loop(0, nblk, pass1,
                             jnp.full((1, G), -1e30, jnp.float32))
        emax = jnp.where(emax < -1e29, 0.0, emax)

        def pass2(bk, c):
            denom, rraw = c
            st = pl.multiple_of(bk * chn, chn)
            bb = b_ref[pl.ds(st, chn), :]
            mff = (bb == gids).astype(jnp.float32)
            ob = out_ref[pl.ds(st, chn), 0:DIM]
            e = e_ref[pl.ds(st, chn), :]
            emax_n = jnp.sum(mff * emax, axis=1, keepdims=True)
            a = jnp.exp(e - emax_n)
            denom = denom + lax.dot_general(
                mff, a, dn, preferred_element_type=jnp.float32,
                precision=lax.Precision.HIGHEST)
            rraw = rraw + lax.dot_general(
                mff, a * ob, dn, preferred_element_type=jnp.float32,
                precision=lax.Precision.HIGHEST)
            return denom, rraw

        denom, rraw = lax.fori_loop(
            0, nblk, pass2,
            (jnp.zeros((G, 1), jnp.float32), jnp.zeros((G, DIM), jnp.float32)))
        r = rraw / (denom + 1e-16)
        qstar = jnp.concatenate([q, r], axis=1)
    z = jnp.maximum(
        jnp.dot(qstar, l1_ref[...], preferred_element_type=jnp.float32,
                precision=lax.Precision.HIGHEST)
        + l1b_ref[...], 0.0)
    o_ref[...] = jnp.dot(z, l2_ref[...], preferred_element_type=jnp.float32,
                         precision=lax.Precision.HIGHEST) + l2b_ref[...]


@functools.cache
def _build_sc_gather():
    mesh = plsc.VectorSubcoreMesh(core_axis_name="c", subcore_axis_name="s")

    @functools.partial(
        pl.kernel,
        mesh=mesh,
        out_type=jax.ShapeDtypeStruct((E2, 128), jnp.float32),
        scratch_types=[
            pltpu.VMEM((NCHUNK, CH), jnp.int32),
            pltpu.VMEM((CH, 128), jnp.float32),
            pltpu.VMEM((CH, 128), jnp.float32),
            pltpu.SemaphoreType.DMA,
            pltpu.SemaphoreType.DMA,
        ],
    )
    def gather_k(nodes_hbm, src2_hbm, xj_hbm, idx_v, buf0, buf1, sem0, sem1):
        wid = lax.axis_index("s") * NC + lax.axis_index("c")
        pltpu.sync_copy(src2_hbm.at[pl.ds(wid * NCHUNK, NCHUNK)], idx_v)
        pending = pltpu.async_copy(nodes_hbm.at[idx_v.at[0]], buf0, sem0)
        for j in range(NCHUNK):
            cbuf = buf0 if j % 2 == 0 else buf1
            nbuf, nsem = (buf1, sem1) if j % 2 == 0 else (buf0, sem0)
            nxt = None
            if j + 1 < NCHUNK:
                nxt = pltpu.async_copy(nodes_hbm.at[idx_v.at[j + 1]], nbuf, nsem)
            pending.wait()
            pltpu.sync_copy(cbuf, xj_hbm.at[pl.ds(wid * EW + j * CH, CH)])
            pending = nxt

    return gather_k


def _sc_gather(nodes, src2):
    return _build_sc_gather()(nodes, src2)


@functools.cache
def _build_sc_scatter():
    mesh = plsc.VectorSubcoreMesh(core_axis_name="c", subcore_axis_name="s")

    @functools.partial(
        pl.kernel,
        mesh=mesh,
        out_type=jax.ShapeDtypeStruct((NC, N2, DIM), jnp.float32),
        scratch_types=[
            pltpu.VMEM_SHARED((N2, DIM), jnp.float32),
            pltpu.VMEM((CH,), jnp.int32),
            pltpu.VMEM((CH, DIM), jnp.float32),
            pltpu.VMEM((CH, DIM), jnp.float32),
            pltpu.SemaphoreType.DMA,
            pltpu.SemaphoreType.DMA,
        ],
        compiler_params=pltpu.CompilerParams(use_tc_tiling_on_sc=False),
    )
    def scatter_k(msg_hbm, dst1_hbm, zeros_hbm, acc_hbm, shared_acc, idx_cur,
                  buf0, buf1, sem0, sem1):
        cid = lax.axis_index("c")
        sid = lax.axis_index("s")
        wid = sid * NC + cid
        pltpu.sync_copy(zeros_hbm.at[pl.ds(sid * SROWS, SROWS)],
                        shared_acc.at[pl.ds(sid * SROWS, SROWS)])
        plsc.subcore_barrier()
        pending = pltpu.async_copy(msg_hbm.at[pl.ds(wid * EW, CH)], buf0, sem0)
        for j in range(NCHUNK):
            cbuf = buf0 if j % 2 == 0 else buf1
            nbuf, nsem = (buf1, sem1) if j % 2 == 0 else (buf0, sem0)
            nxt = None
            if j + 1 < NCHUNK:
                nxt = pltpu.async_copy(
                    msg_hbm.at[pl.ds(wid * EW + (j + 1) * CH, CH)], nbuf, nsem)
            pltpu.sync_copy(dst1_hbm.at[pl.ds((wid * NCHUNK + j) * CH, CH)],
                            idx_cur)
            pending.wait()
            pltpu.sync_copy(cbuf, shared_acc.at[idx_cur], add=True)
            pending = nxt
        plsc.subcore_barrier()
        pltpu.sync_copy(shared_acc.at[pl.ds(sid * SROWS, SROWS)],
                        acc_hbm.at[cid].at[pl.ds(sid * SROWS, SROWS)])

    return scatter_k


def _sc_scatter(msg, dst2, zeros_n):
    return _build_sc_scatter()(msg, dst2, zeros_n)


@functools.cache
def _build_sc_deg():
    mesh = plsc.VectorSubcoreMesh(core_axis_name="c", subcore_axis_name="s")

    @functools.partial(
        pl.kernel,
        mesh=mesh,
        out_type=jax.ShapeDtypeStruct((NC, N2, DIM), jnp.float32),
        scratch_types=[
            pltpu.VMEM_SHARED((N2, DIM), jnp.float32),
            pltpu.VMEM((CH,), jnp.int32),
            pltpu.VMEM((CH, DIM), jnp.float32),
        ],
        compiler_params=pltpu.CompilerParams(use_tc_tiling_on_sc=False),
    )
    def deg_k(dst1_hbm, ones_hbm, zeros_hbm, deg_hbm, shared_acc, idx_cur, buf0):
        cid = lax.axis_index("c")
        sid = lax.axis_index("s")
        wid = sid * NC + cid
        pltpu.sync_copy(zeros_hbm.at[pl.ds(sid * SROWS, SROWS)],
                        shared_acc.at[pl.ds(sid * SROWS, SROWS)])
        pltpu.sync_copy(ones_hbm, buf0)
        plsc.subcore_barrier()
        for j in range(NCHUNK):
            pltpu.sync_copy(dst1_hbm.at[pl.ds((wid * NCHUNK + j) * CH, CH)],
                            idx_cur)
            if j < REAL_CHUNKS_LAST:
                pltpu.sync_copy(buf0, shared_acc.at[idx_cur], add=True)
            else:
                @pl.when(wid != NW - 1)
                def _():
                    pltpu.sync_copy(buf0, shared_acc.at[idx_cur], add=True)
        plsc.subcore_barrier()
        pltpu.sync_copy(shared_acc.at[pl.ds(sid * SROWS, SROWS)],
                        deg_hbm.at[cid].at[pl.ds(sid * SROWS, SROWS)])

    return deg_k


def _sc_deg(dst2, ones_ch, zeros_n):
    return _build_sc_deg()(dst2, ones_ch, zeros_n)


def kernel(x, edge_index, edge_attr, batch, lin0_W, lin0_b, nn1_W, nn1_b,
           nn2_W, nn2_b, conv_root, conv_bias, gru_W_ih, gru_W_hh, gru_b_ih,
           gru_b_hh, lstm_W_ih, lstm_W_hh, lstm_b_ih, lstm_b_hh, lin1_W,
           lin1_b, lin2_W, lin2_b):
    f32 = jnp.float32
    pad = E2 - E
    src2 = jnp.concatenate(
        [edge_index[0].astype(jnp.int32), jnp.zeros((pad,), jnp.int32)]
    ).reshape(NW * NCHUNK, CH)
    dst1 = jnp.concatenate(
        [edge_index[1].astype(jnp.int32), jnp.zeros((pad,), jnp.int32)])
    ea_p = jnp.zeros((E2, 8), f32).at[:E, :5].set(edge_attr)
    x_p = jnp.zeros((N, 16), f32).at[:, :11].set(x)
    batch2 = batch.astype(jnp.int32).reshape(N, 1)

    w_l0 = jnp.zeros((16, 128), f32).at[:11, :DIM].set(lin0_W.T)
    b_l0 = jnp.zeros((1, 128), f32).at[:, :DIM].set(lin0_b.reshape(1, DIM))
    w1 = jnp.zeros((8, 128), f32).at[:5].set(nn1_W.T)
    b1 = nn1_b.reshape(1, 128)
    w2 = nn2_W.T
    b2 = nn2_b.reshape(1, DIM * DIM)
    cb = conv_bias.reshape(1, DIM)
    gwih = gru_W_ih.T
    gwhh = gru_W_hh.T
    gbih = gru_b_ih.reshape(1, 3 * DIM)
    gbhh = gru_b_hh.reshape(1, 3 * DIM)
    lwih = lstm_W_ih.T
    lwhh = lstm_W_hh.T
    lbih = lstm_b_ih.reshape(1, 4 * DIM)
    lbhh = lstm_b_hh.reshape(1, 4 * DIM)
    l1 = lin1_W.T
    l1b = lin1_b.reshape(1, DIM)
    l2 = lin2_W.T
    l2b = lin2_b.reshape(1, 1)

    zeros_n = jnp.zeros((N2, DIM), f32)
    ones_ch = jnp.ones((CH, DIM), f32)

    out = pl.pallas_call(
        _lin0_body,
        grid=(N // NBLK,),
        in_specs=[
            pl.BlockSpec((NBLK, 16), lambda i: (i, 0)),
            pl.BlockSpec((16, 128), lambda i: (0, 0)),
            pl.BlockSpec((1, 128), lambda i: (0, 0)),
        ],
        out_specs=pl.BlockSpec((NBLK, 128), lambda i: (i, 0)),
        out_shape=jax.ShapeDtypeStruct((N, 128), f32),
    )(x_p, w_l0, b_l0)

    degp = _sc_deg(dst1, ones_ch, zeros_n)

    edge_call = pl.pallas_call(
        _edge_body,
        grid=(E2 // EBLK,),
        in_specs=[
            pl.BlockSpec((EBLK, 8), lambda i: (i, 0)),
            pl.BlockSpec((EBLK, 128), lambda i: (i, 0)),
            pl.BlockSpec((8, 128), lambda i: (0, 0)),
            pl.BlockSpec((1, 128), lambda i: (0, 0)),
            pl.BlockSpec((128, DIM * DIM), lambda i: (0, 0)),
            pl.BlockSpec((1, DIM * DIM), lambda i: (0, 0)),
        ],
        out_specs=pl.BlockSpec((EBLK, DIM), lambda i: (i, 0)),
        out_shape=jax.ShapeDtypeStruct((E2, DIM), f32),
    )

    node_call = pl.pallas_call(
        _node_body,
        grid=(N // NBLK,),
        in_specs=[
            pl.BlockSpec((NC, NBLK, DIM), lambda i: (0, i, 0)),
            pl.BlockSpec((NC, NBLK, DIM), lambda i: (0, i, 0)),
            pl.BlockSpec((NBLK, 128), lambda i: (i, 0)),
            pl.BlockSpec((DIM, DIM), lambda i: (0, 0)),
            pl.BlockSpec((1, DIM), lambda i: (0, 0)),
            pl.BlockSpec((DIM, 3 * DIM), lambda i: (0, 0)),
            pl.BlockSpec((DIM, 3 * DIM), lambda i: (0, 0)),
            pl.BlockSpec((1, 3 * DIM), lambda i: (0, 0)),
            pl.BlockSpec((1, 3 * DIM), lambda i: (0, 0)),
        ],
        out_specs=pl.BlockSpec((NBLK, 128), lambda i: (i, 0)),
        out_shape=jax.ShapeDtypeStruct((N, 128), f32),
    )

    for _ in range(3):
        xj = _sc_gather(out, src2)
        msg = edge_call(ea_p, xj, w1, b1, w2, b2)
        aggp = _sc_scatter(msg, dst1, zeros_n)
        out = node_call(aggp, degp, out, conv_root, cb, gwih, gwhh, gbih, gbhh)

    z = pl.pallas_call(
        _s2s_body,
        out_shape=jax.ShapeDtypeStruct((G, 1), f32),
        scratch_shapes=[pltpu.VMEM((N, 1), f32)],
    )(out, batch2, lwih, lwhh, lbih, lbhh, l1, l1b, l2, l2b)
    return z.reshape(-1)


# speedup vs baseline: 2.2323x; 2.2323x over previous
import functools

import jax
import jax.numpy as jnp
from jax import lax
from jax.experimental import pallas as pl
from jax.experimental.pallas import tpu as pltpu
from jax.experimental.pallas import tpu_sc as plsc

DIM = 32
N = 10000
E = 160000
G = 500

NC, NS = 2, 16
NW = NC * NS
CH = 128
NCHUNK = 40
EW = CH * NCHUNK
E2 = EW * NW
REAL_CHUNKS_LAST = (E - (NW - 1) * EW) // CH

EBLK = 1024
NBLK = 2000
N2 = 10240
SROWS = N2 // NS


def _lin0_body(x_ref, w_ref, b_ref, o_ref):
    o_ref[...] = jnp.maximum(
        jnp.dot(x_ref[...], w_ref[...], preferred_element_type=jnp.float32,
                precision=lax.Precision.HIGHEST)
        + b_ref[...], 0.0)


def _edge_body(ea_ref, xj_ref, w1_ref, b1_ref, w2_ref, b2_ref, bsel_ref,
               rsel_ref, o_ref):
    hi = lax.Precision.DEFAULT
    eh = jnp.maximum(
        jnp.dot(ea_ref[...], w1_ref[...], preferred_element_type=jnp.float32,
                precision=hi)
        + b1_ref[...], 0.0)
    theta = jnp.dot(eh, w2_ref[...], preferred_element_type=jnp.float32,
                    precision=hi) + b2_ref[...]
    xj = xj_ref[:, 0:DIM]
    x_rep = jnp.dot(xj, bsel_ref[...], preferred_element_type=jnp.float32,
                    precision=hi)
    acc = jnp.dot(theta * x_rep, rsel_ref[...],
                  preferred_element_type=jnp.float32, precision=hi)
    row = pl.program_id(0) * EBLK + lax.broadcasted_iota(jnp.int32, (EBLK, 1), 0)
    o_ref[...] = jnp.where(row < E, acc, 0.0)


def _node_body(aggp_ref, degp_ref, out_ref, cr_ref, cb_ref, wih_ref, whh_ref,
               bih_ref, bhh_ref, o_ref):
    agg = aggp_ref[0] + aggp_ref[1]
    deg = jnp.maximum(degp_ref[0] + degp_ref[1], 1.0)
    h = out_ref[:, 0:DIM]
    m = jnp.maximum(
        agg / deg
        + jnp.dot(h, cr_ref[...], preferred_element_type=jnp.float32,
                  precision=lax.Precision.HIGHEST)
        + cb_ref[...], 0.0)
    gi = jnp.dot(m, wih_ref[...], preferred_element_type=jnp.float32,
                 precision=lax.Precision.HIGHEST) + bih_ref[...]
    gh = jnp.dot(h, whh_ref[...], preferred_element_type=jnp.float32,
                 precision=lax.Precision.HIGHEST) + bhh_ref[...]
    r = jax.nn.sigmoid(gi[:, 0:DIM] + gh[:, 0:DIM])
    z = jax.nn.sigmoid(gi[:, DIM:2 * DIM] + gh[:, DIM:2 * DIM])
    n = jnp.tanh(gi[:, 2 * DIM:] + r * gh[:, 2 * DIM:])
    o_ref[...] = jnp.concatenate(
        [(1.0 - z) * n + z * h, jnp.zeros((NBLK, 128 - DIM), jnp.float32)], axis=1)


def _s2s_body(out_ref, b_ref, wih_ref, whh_ref, bih_ref, bhh_ref,
              l1_ref, l1b_ref, l2_ref, l2b_ref, o_ref, e_ref):
    nblk = 10
    chn = N // nblk
    gids = lax.broadcasted_iota(jnp.int32, (1, G), 1)
    dn = (((0,), (0,)), ((), ()))
    qstar = jnp.zeros((G, 2 * DIM), jnp.float32)
    hh = jnp.zeros((G, DIM), jnp.float32)
    cc = jnp.zeros((G, DIM), jnp.float32)
    for _ in range(3):
        g = (jnp.dot(qstar, wih_ref[...], preferred_element_type=jnp.float32,
                     precision=lax.Precision.HIGHEST)
             + bih_ref[...]
             + jnp.dot(hh, whh_ref[...], preferred_element_type=jnp.float32,
                       precision=lax.Precision.HIGHEST)
             + bhh_ref[...])
        ii = jax.nn.sigmoid(g[:, 0:DIM])
        ff = jax.nn.sigmoid(g[:, DIM:2 * DIM])
        gg = jnp.tanh(g[:, 2 * DIM:3 * DIM])
        oo = jax.nn.sigmoid(g[:, 3 * DIM:])
        cc = ff * cc + ii * gg
        hh = oo * jnp.tanh(cc)
        q = hh

        def pass1(bk, emax):
            st = pl.multiple_of(bk * chn, chn)
            bb = b_ref[pl.ds(st, chn), :]
            mff = (bb == gids).astype(jnp.float32)
            ob = out_ref[pl.ds(st, chn), 0:DIM]
            qg = jnp.dot(mff, q, preferred_element_type=jnp.float32,
                         precision=lax.Precision.HIGHEST)
            e = jnp.sum(ob * qg, axis=1, keepdims=True)
            e_ref[pl.ds(st, chn), :] = e
            return jnp.maximum(
                emax,
                jnp.max(mff * e + (mff - 1.0) * 1e30, axis=0, keepdims=True))

        emax = lax.fori_loop(0, nblk, pass1,
                             jnp.full((1, G), -1e30, jnp.float32))
        emax = jnp.where(emax < -1e29, 0.0, emax)

        def pass2(bk, c):
            denom, rraw = c
            st = pl.multiple_of(bk * chn, chn)
            bb = b_ref[pl.ds(st, chn), :]
            mff = (bb == gids).astype(jnp.float32)
            ob = out_ref[pl.ds(st, chn), 0:DIM]
            e = e_ref[pl.ds(st, chn), :]
            emax_n = jnp.sum(mff * emax, axis=1, keepdims=True)
            a = jnp.exp(e - emax_n)
            denom = denom + lax.dot_general(
                mff, a, dn, preferred_element_type=jnp.float32,
                precision=lax.Precision.HIGHEST)
            rraw = rraw + lax.dot_general(
                mff, a * ob, dn, preferred_element_type=jnp.float32,
                precision=lax.Precision.HIGHEST)
            return denom, rraw

        denom, rraw = lax.fori_loop(
            0, nblk, pass2,
            (jnp.zeros((G, 1), jnp.float32), jnp.zeros((G, DIM), jnp.float32)))
        r = rraw / (denom + 1e-16)
        qstar = jnp.concatenate([q, r], axis=1)
    z = jnp.maximum(
        jnp.dot(qstar, l1_ref[...], preferred_element_type=jnp.float32,
                precision=lax.Precision.HIGHEST)
        + l1b_ref[...], 0.0)
    o_ref[...] = jnp.dot(z, l2_ref[...], preferred_element_type=jnp.float32,
                         precision=lax.Precision.HIGHEST) + l2b_ref[...]


@functools.cache
def _build_sc_gather():
    mesh = plsc.VectorSubcoreMesh(core_axis_name="c", subcore_axis_name="s")

    @functools.partial(
        pl.kernel,
        mesh=mesh,
        out_type=jax.ShapeDtypeStruct((E2, 128), jnp.float32),
        scratch_types=[
            pltpu.VMEM((NCHUNK, CH), jnp.int32),
            pltpu.VMEM((CH, 128), jnp.float32),
            pltpu.VMEM((CH, 128), jnp.float32),
            pltpu.SemaphoreType.DMA,
            pltpu.SemaphoreType.DMA,
        ],
    )
    def gather_k(nodes_hbm, src2_hbm, xj_hbm, idx_v, buf0, buf1, sem0, sem1):
        wid = lax.axis_index("s") * NC + lax.axis_index("c")
        pltpu.sync_copy(src2_hbm.at[pl.ds(wid * NCHUNK, NCHUNK)], idx_v)
        pending = pltpu.async_copy(nodes_hbm.at[idx_v.at[0]], buf0, sem0)
        for j in range(NCHUNK):
            cbuf = buf0 if j % 2 == 0 else buf1
            nbuf, nsem = (buf1, sem1) if j % 2 == 0 else (buf0, sem0)
            nxt = None
            if j + 1 < NCHUNK:
                nxt = pltpu.async_copy(nodes_hbm.at[idx_v.at[j + 1]], nbuf, nsem)
            pending.wait()
            pltpu.sync_copy(cbuf, xj_hbm.at[pl.ds(wid * EW + j * CH, CH)])
            pending = nxt

    return gather_k


def _sc_gather(nodes, src2):
    return _build_sc_gather()(nodes, src2)


@functools.cache
def _build_sc_scatter():
    mesh = plsc.VectorSubcoreMesh(core_axis_name="c", subcore_axis_name="s")

    @functools.partial(
        pl.kernel,
        mesh=mesh,
        out_type=jax.ShapeDtypeStruct((NC, N2, DIM), jnp.float32),
        scratch_types=[
            pltpu.VMEM_SHARED((N2, DIM), jnp.float32),
            pltpu.VMEM((CH,), jnp.int32),
            pltpu.VMEM((CH, DIM), jnp.float32),
            pltpu.VMEM((CH, DIM), jnp.float32),
            pltpu.SemaphoreType.DMA,
            pltpu.SemaphoreType.DMA,
        ],
        compiler_params=pltpu.CompilerParams(use_tc_tiling_on_sc=False),
    )
    def scatter_k(msg_hbm, dst1_hbm, zeros_hbm, acc_hbm, shared_acc, idx_cur,
                  buf0, buf1, sem0, sem1):
        cid = lax.axis_index("c")
        sid = lax.axis_index("s")
        wid = sid * NC + cid
        pltpu.sync_copy(zeros_hbm.at[pl.ds(sid * SROWS, SROWS)],
                        shared_acc.at[pl.ds(sid * SROWS, SROWS)])
        plsc.subcore_barrier()
        pending = pltpu.async_copy(msg_hbm.at[pl.ds(wid * EW, CH)], buf0, sem0)
        for j in range(NCHUNK):
            cbuf = buf0 if j % 2 == 0 else buf1
            nbuf, nsem = (buf1, sem1) if j % 2 == 0 else (buf0, sem0)
            nxt = None
            if j + 1 < NCHUNK:
                nxt = pltpu.async_copy(
                    msg_hbm.at[pl.ds(wid * EW + (j + 1) * CH, CH)], nbuf, nsem)
            pltpu.sync_copy(dst1_hbm.at[pl.ds((wid * NCHUNK + j) * CH, CH)],
                            idx_cur)
            pending.wait()
            pltpu.sync_copy(cbuf, shared_acc.at[idx_cur], add=True)
            pending = nxt
        plsc.subcore_barrier()
        pltpu.sync_copy(shared_acc.at[pl.ds(sid * SROWS, SROWS)],
                        acc_hbm.at[cid].at[pl.ds(sid * SROWS, SROWS)])

    return scatter_k


def _sc_scatter(msg, dst2, zeros_n):
    return _build_sc_scatter()(msg, dst2, zeros_n)


@functools.cache
def _build_sc_deg():
    mesh = plsc.VectorSubcoreMesh(core_axis_name="c", subcore_axis_name="s")

    @functools.partial(
        pl.kernel,
        mesh=mesh,
        out_type=jax.ShapeDtypeStruct((NC, N2, DIM), jnp.float32),
        scratch_types=[
            pltpu.VMEM_SHARED((N2, DIM), jnp.float32),
            pltpu.VMEM((CH,), jnp.int32),
            pltpu.VMEM((CH, DIM), jnp.float32),
        ],
        compiler_params=pltpu.CompilerParams(use_tc_tiling_on_sc=False),
    )
    def deg_k(dst1_hbm, ones_hbm, zeros_hbm, deg_hbm, shared_acc, idx_cur, buf0):
        cid = lax.axis_index("c")
        sid = lax.axis_index("s")
        wid = sid * NC + cid
        pltpu.sync_copy(zeros_hbm.at[pl.ds(sid * SROWS, SROWS)],
                        shared_acc.at[pl.ds(sid * SROWS, SROWS)])
        pltpu.sync_copy(ones_hbm, buf0)
        plsc.subcore_barrier()
        for j in range(NCHUNK):
            pltpu.sync_copy(dst1_hbm.at[pl.ds((wid * NCHUNK + j) * CH, CH)],
                            idx_cur)
            if j < REAL_CHUNKS_LAST:
                pltpu.sync_copy(buf0, shared_acc.at[idx_cur], add=True)
            else:
                @pl.when(wid != NW - 1)
                def _():
                    pltpu.sync_copy(buf0, shared_acc.at[idx_cur], add=True)
        plsc.subcore_barrier()
        pltpu.sync_copy(shared_acc.at[pl.ds(sid * SROWS, SROWS)],
                        deg_hbm.at[cid].at[pl.ds(sid * SROWS, SROWS)])

    return deg_k


def _sc_deg(dst2, ones_ch, zeros_n):
    return _build_sc_deg()(dst2, ones_ch, zeros_n)


def kernel(x, edge_index, edge_attr, batch, lin0_W, lin0_b, nn1_W, nn1_b,
           nn2_W, nn2_b, conv_root, conv_bias, gru_W_ih, gru_W_hh, gru_b_ih,
           gru_b_hh, lstm_W_ih, lstm_W_hh, lstm_b_ih, lstm_b_hh, lin1_W,
           lin1_b, lin2_W, lin2_b):
    f32 = jnp.float32
    pad = E2 - E
    src2 = jnp.concatenate(
        [edge_index[0].astype(jnp.int32), jnp.zeros((pad,), jnp.int32)]
    ).reshape(NW * NCHUNK, CH)
    dst1 = jnp.concatenate(
        [edge_index[1].astype(jnp.int32), jnp.zeros((pad,), jnp.int32)])
    ea_p = jnp.zeros((E2, 8), f32).at[:E, :5].set(edge_attr)
    x_p = jnp.zeros((N, 16), f32).at[:, :11].set(x)
    batch2 = batch.astype(jnp.int32).reshape(N, 1)

    w_l0 = jnp.zeros((16, 128), f32).at[:11, :DIM].set(lin0_W.T)
    b_l0 = jnp.zeros((1, 128), f32).at[:, :DIM].set(lin0_b.reshape(1, DIM))
    w1 = jnp.zeros((8, 128), f32).at[:5].set(nn1_W.T)
    b1 = nn1_b.reshape(1, 128)
    w2 = nn2_W.T
    b2 = nn2_b.reshape(1, DIM * DIM)
    cb = conv_bias.reshape(1, DIM)
    gwih = gru_W_ih.T
    gwhh = gru_W_hh.T
    gbih = gru_b_ih.reshape(1, 3 * DIM)
    gbhh = gru_b_hh.reshape(1, 3 * DIM)
    lwih = lstm_W_ih.T
    lwhh = lstm_W_hh.T
    lbih = lstm_b_ih.reshape(1, 4 * DIM)
    lbhh = lstm_b_hh.reshape(1, 4 * DIM)
    l1 = lin1_W.T
    l1b = lin1_b.reshape(1, DIM)
    l2 = lin2_W.T
    l2b = lin2_b.reshape(1, 1)

    bsel = jnp.repeat(jnp.eye(DIM, dtype=f32), DIM, axis=1)
    rsel = jnp.tile(jnp.eye(DIM, dtype=f32), (DIM, 1))
    zeros_n = jnp.zeros((N2, DIM), f32)
    ones_ch = jnp.ones((CH, DIM), f32)

    out = pl.pallas_call(
        _lin0_body,
        grid=(N // NBLK,),
        in_specs=[
            pl.BlockSpec((NBLK, 16), lambda i: (i, 0)),
            pl.BlockSpec((16, 128), lambda i: (0, 0)),
            pl.BlockSpec((1, 128), lambda i: (0, 0)),
        ],
        out_specs=pl.BlockSpec((NBLK, 128), lambda i: (i, 0)),
        out_shape=jax.ShapeDtypeStruct((N, 128), f32),
    )(x_p, w_l0, b_l0)

    degp = _sc_deg(dst1, ones_ch, zeros_n)

    edge_call = pl.pallas_call(
        _edge_body,
        grid=(E2 // EBLK,),
        in_specs=[
            pl.BlockSpec((EBLK, 8), lambda i: (i, 0)),
            pl.BlockSpec((EBLK, 128), lambda i: (i, 0)),
            pl.BlockSpec((8, 128), lambda i: (0, 0)),
            pl.BlockSpec((1, 128), lambda i: (0, 0)),
            pl.BlockSpec((128, DIM * DIM), lambda i: (0, 0)),
            pl.BlockSpec((1, DIM * DIM), lambda i: (0, 0)),
            pl.BlockSpec((DIM, DIM * DIM), lambda i: (0, 0)),
            pl.BlockSpec((DIM * DIM, DIM), lambda i: (0, 0)),
        ],
        out_specs=pl.BlockSpec((EBLK, DIM), lambda i: (i, 0)),
        out_shape=jax.ShapeDtypeStruct((E2, DIM), f32),
    )

    node_call = pl.pallas_call(
        _node_body,
        grid=(N // NBLK,),
        in_specs=[
            pl.BlockSpec((NC, NBLK, DIM), lambda i: (0, i, 0)),
            pl.BlockSpec((NC, NBLK, DIM), lambda i: (0, i, 0)),
            pl.BlockSpec((NBLK, 128), lambda i: (i, 0)),
            pl.BlockSpec((DIM, DIM), lambda i: (0, 0)),
            pl.BlockSpec((1, DIM), lambda i: (0, 0)),
            pl.BlockSpec((DIM, 3 * DIM), lambda i: (0, 0)),
            pl.BlockSpec((DIM, 3 * DIM), lambda i: (0, 0)),
            pl.BlockSpec((1, 3 * DIM), lambda i: (0, 0)),
            pl.BlockSpec((1, 3 * DIM), lambda i: (0, 0)),
        ],
        out_specs=pl.BlockSpec((NBLK, 128), lambda i: (i, 0)),
        out_shape=jax.ShapeDtypeStruct((N, 128), f32),
    )

    for _ in range(3):
        xj = _sc_gather(out, src2)
        msg = edge_call(ea_p, xj, w1, b1, w2, b2, bsel, rsel)
        aggp = _sc_scatter(msg, dst1, zeros_n)
        out = node_call(aggp, degp, out, conv_root, cb, gwih, gwhh, gbih, gbhh)

    z = pl.pallas_call(
        _s2s_body,
        out_shape=jax.ShapeDtypeStruct((G, 1), f32),
        scratch_shapes=[pltpu.VMEM((N, 1), f32)],
    )(out, batch2, lwih, lwhh, lbih, lbhh, l1, l1b, l2, l2b)
    return z.reshape(-1)


# speedup vs baseline: 2.5997x; 1.1646x over previous
import functools

import jax
import jax.numpy as jnp
from jax import lax
from jax.experimental import pallas as pl
from jax.experimental.pallas import tpu as pltpu
from jax.experimental.pallas import tpu_sc as plsc

DIM = 32
N = 10000
E = 160000
G = 500

NC, NS = 2, 16
NW = NC * NS
CH = 128
NCHUNK = 40
EW = CH * NCHUNK
E2 = EW * NW
REAL_CHUNKS_LAST = (E - (NW - 1) * EW) // CH

EBLK = 1024
NBLK = 2000
N2 = 10240
SROWS = N2 // NS


def _lin0_body(x_ref, w_ref, b_ref, o_ref):
    o_ref[...] = jnp.maximum(
        jnp.dot(x_ref[...], w_ref[...], preferred_element_type=jnp.float32,
                precision=lax.Precision.HIGHEST)
        + b_ref[...], 0.0)


def _edge_body(ea_ref, xj_ref, w1_ref, b1_ref, w2_ref, b2_ref, bsel_ref,
               rsel_ref, o_ref):
    hi = lax.Precision.DEFAULT
    eh = jnp.maximum(
        jnp.dot(ea_ref[...], w1_ref[...], preferred_element_type=jnp.float32,
                precision=hi)
        + b1_ref[...], 0.0)
    theta = jnp.dot(eh, w2_ref[...], preferred_element_type=jnp.float32,
                    precision=hi) + b2_ref[...]
    xj = xj_ref[...]
    x_rep = jnp.dot(xj, bsel_ref[...], preferred_element_type=jnp.float32,
                    precision=hi)
    acc = jnp.dot(theta * x_rep, rsel_ref[...],
                  preferred_element_type=jnp.float32, precision=hi)
    row = pl.program_id(0) * EBLK + lax.broadcasted_iota(jnp.int32, (EBLK, 1), 0)
    o_ref[...] = jnp.where(row < E, acc, 0.0)


def _node_body(aggp_ref, degp_ref, out_ref, cr_ref, cb_ref, wih_ref, whh_ref,
               bih_ref, bhh_ref, o_ref):
    agg = aggp_ref[0] + aggp_ref[1]
    deg = jnp.maximum(degp_ref[0] + degp_ref[1], 1.0)
    h = out_ref[...]
    m = jnp.maximum(
        agg / deg
        + jnp.dot(h, cr_ref[...], preferred_element_type=jnp.float32,
                  precision=lax.Precision.HIGHEST)
        + cb_ref[...], 0.0)
    gi = jnp.dot(m, wih_ref[...], preferred_element_type=jnp.float32,
                 precision=lax.Precision.HIGHEST) + bih_ref[...]
    gh = jnp.dot(h, whh_ref[...], preferred_element_type=jnp.float32,
                 precision=lax.Precision.HIGHEST) + bhh_ref[...]
    r = jax.nn.sigmoid(gi[:, 0:DIM] + gh[:, 0:DIM])
    z = jax.nn.sigmoid(gi[:, DIM:2 * DIM] + gh[:, DIM:2 * DIM])
    n = jnp.tanh(gi[:, 2 * DIM:] + r * gh[:, 2 * DIM:])
    o_ref[...] = (1.0 - z) * n + z * h


def _s2s_body(out_ref, b_ref, wih_ref, whh_ref, bih_ref, bhh_ref,
              l1_ref, l1b_ref, l2_ref, l2b_ref, o_ref, e_ref):
    nblk = 10
    chn = N // nblk
    gids = lax.broadcasted_iota(jnp.int32, (1, G), 1)
    dn = (((0,), (0,)), ((), ()))
    qstar = jnp.zeros((G, 2 * DIM), jnp.float32)
    hh = jnp.zeros((G, DIM), jnp.float32)
    cc = jnp.zeros((G, DIM), jnp.float32)
    for _ in range(3):
        g = (jnp.dot(qstar, wih_ref[...], preferred_element_type=jnp.float32,
                     precision=lax.Precision.HIGHEST)
             + bih_ref[...]
             + jnp.dot(hh, whh_ref[...], preferred_element_type=jnp.float32,
                       precision=lax.Precision.HIGHEST)
             + bhh_ref[...])
        ii = jax.nn.sigmoid(g[:, 0:DIM])
        ff = jax.nn.sigmoid(g[:, DIM:2 * DIM])
        gg = jnp.tanh(g[:, 2 * DIM:3 * DIM])
        oo = jax.nn.sigmoid(g[:, 3 * DIM:])
        cc = ff * cc + ii * gg
        hh = oo * jnp.tanh(cc)
        q = hh

        def pass1(bk, emax):
            st = pl.multiple_of(bk * chn, chn)
            bb = b_ref[pl.ds(st, chn), :]
            mff = (bb == gids).astype(jnp.float32)
            ob = out_ref[pl.ds(st, chn), 0:DIM]
            qg = jnp.dot(mff, q, preferred_element_type=jnp.float32,
                         precision=lax.Precision.HIGHEST)
            e = jnp.sum(ob * qg, axis=1, keepdims=True)
            e_ref[pl.ds(st, chn), :] = e
            return jnp.maximum(
                emax,
                jnp.max(mff * e + (mff - 1.0) * 1e30, axis=0, keepdims=True))

        emax = lax.fori_loop(0, nblk, pass1,
                             jnp.full((1, G), -1e30, jnp.float32))
        emax = jnp.where(emax < -1e29, 0.0, emax)

        def pass2(bk, c):
            denom, rraw = c
            st = pl.multiple_of(bk * chn, chn)
            bb = b_ref[pl.ds(st, chn), :]
            mff = (bb == gids).astype(jnp.float32)
            ob = out_ref[pl.ds(st, chn), 0:DIM]
            e = e_ref[pl.ds(st, chn), :]
            emax_n = jnp.sum(mff * emax, axis=1, keepdims=True)
            a = jnp.exp(e - emax_n)
            denom = denom + lax.dot_general(
                mff, a, dn, preferred_element_type=jnp.float32,
                precision=lax.Precision.HIGHEST)
            rraw = rraw + lax.dot_general(
                mff, a * ob, dn, preferred_element_type=jnp.float32,
                precision=lax.Precision.HIGHEST)
            return denom, rraw

        denom, rraw = lax.fori_loop(
            0, nblk, pass2,
            (jnp.zeros((G, 1), jnp.float32), jnp.zeros((G, DIM), jnp.float32)))
        r = rraw / (denom + 1e-16)
        qstar = jnp.concatenate([q, r], axis=1)
    z = jnp.maximum(
        jnp.dot(qstar, l1_ref[...], preferred_element_type=jnp.float32,
                precision=lax.Precision.HIGHEST)
        + l1b_ref[...], 0.0)
    o_ref[...] = jnp.dot(z, l2_ref[...], preferred_element_type=jnp.float32,
                         precision=lax.Precision.HIGHEST) + l2b_ref[...]


@functools.cache
def _build_sc_gather():
    mesh = plsc.VectorSubcoreMesh(core_axis_name="c", subcore_axis_name="s")

    @functools.partial(
        pl.kernel,
        mesh=mesh,
        out_type=jax.ShapeDtypeStruct((E2, DIM), jnp.float32),
        scratch_types=[
            pltpu.VMEM((NCHUNK, CH), jnp.int32),
            pltpu.VMEM((CH, DIM), jnp.float32),
            pltpu.VMEM((CH, DIM), jnp.float32),
            pltpu.SemaphoreType.DMA,
            pltpu.SemaphoreType.DMA,
        ],
        compiler_params=pltpu.CompilerParams(use_tc_tiling_on_sc=False),
    )
    def gather_k(nodes_hbm, src2_hbm, xj_hbm, idx_v, buf0, buf1, sem0, sem1):
        wid = lax.axis_index("s") * NC + lax.axis_index("c")
        pltpu.sync_copy(src2_hbm.at[pl.ds(wid * NCHUNK, NCHUNK)], idx_v)
        pending = pltpu.async_copy(nodes_hbm.at[idx_v.at[0]], buf0, sem0)
        for j in range(NCHUNK):
            cbuf = buf0 if j % 2 == 0 else buf1
            nbuf, nsem = (buf1, sem1) if j % 2 == 0 else (buf0, sem0)
            nxt = None
            if j + 1 < NCHUNK:
                nxt = pltpu.async_copy(nodes_hbm.at[idx_v.at[j + 1]], nbuf, nsem)
            pending.wait()
            pltpu.sync_copy(cbuf, xj_hbm.at[pl.ds(wid * EW + j * CH, CH)])
            pending = nxt

    return gather_k


def _sc_gather(nodes, src2):
    return _build_sc_gather()(nodes, src2)


@functools.cache
def _build_sc_scatter():
    mesh = plsc.VectorSubcoreMesh(core_axis_name="c", subcore_axis_name="s")

    @functools.partial(
        pl.kernel,
        mesh=mesh,
        out_type=jax.ShapeDtypeStruct((NC, N2, DIM), jnp.float32),
        scratch_types=[
            pltpu.VMEM_SHARED((N2, DIM), jnp.float32),
            pltpu.VMEM((CH,), jnp.int32),
            pltpu.VMEM((CH, DIM), jnp.float32),
            pltpu.VMEM((CH, DIM), jnp.float32),
            pltpu.SemaphoreType.DMA,
            pltpu.SemaphoreType.DMA,
        ],
        compiler_params=pltpu.CompilerParams(use_tc_tiling_on_sc=False),
    )
    def scatter_k(msg_hbm, dst1_hbm, zeros_hbm, acc_hbm, shared_acc, idx_cur,
                  buf0, buf1, sem0, sem1):
        cid = lax.axis_index("c")
        sid = lax.axis_index("s")
        wid = sid * NC + cid
        pltpu.sync_copy(zeros_hbm.at[pl.ds(sid * SROWS, SROWS)],
                        shared_acc.at[pl.ds(sid * SROWS, SROWS)])
        plsc.subcore_barrier()
        pending = pltpu.async_copy(msg_hbm.at[pl.ds(wid * EW, CH)], buf0, sem0)
        for j in range(NCHUNK):
            cbuf = buf0 if j % 2 == 0 else buf1
            nbuf, nsem = (buf1, sem1) if j % 2 == 0 else (buf0, sem0)
            nxt = None
            if j + 1 < NCHUNK:
                nxt = pltpu.async_copy(
                    msg_hbm.at[pl.ds(wid * EW + (j + 1) * CH, CH)], nbuf, nsem)
            pltpu.sync_copy(dst1_hbm.at[pl.ds((wid * NCHUNK + j) * CH, CH)],
                            idx_cur)
            pending.wait()
            pltpu.sync_copy(cbuf, shared_acc.at[idx_cur], add=True)
            pending = nxt
        plsc.subcore_barrier()
        pltpu.sync_copy(shared_acc.at[pl.ds(sid * SROWS, SROWS)],
                        acc_hbm.at[cid].at[pl.ds(sid * SROWS, SROWS)])

    return scatter_k


def _sc_scatter(msg, dst2, zeros_n):
    return _build_sc_scatter()(msg, dst2, zeros_n)


@functools.cache
def _build_sc_deg():
    mesh = plsc.VectorSubcoreMesh(core_axis_name="c", subcore_axis_name="s")

    @functools.partial(
        pl.kernel,
        mesh=mesh,
        out_type=jax.ShapeDtypeStruct((NC, N2, DIM), jnp.float32),
        scratch_types=[
            pltpu.VMEM_SHARED((N2, DIM), jnp.float32),
            pltpu.VMEM((CH,), jnp.int32),
            pltpu.VMEM((CH, DIM), jnp.float32),
        ],
        compiler_params=pltpu.CompilerParams(use_tc_tiling_on_sc=False),
    )
    def deg_k(dst1_hbm, ones_hbm, zeros_hbm, deg_hbm, shared_acc, idx_cur, buf0):
        cid = lax.axis_index("c")
        sid = lax.axis_index("s")
        wid = sid * NC + cid
        pltpu.sync_copy(zeros_hbm.at[pl.ds(sid * SROWS, SROWS)],
                        shared_acc.at[pl.ds(sid * SROWS, SROWS)])
        pltpu.sync_copy(ones_hbm, buf0)
        plsc.subcore_barrier()
        for j in range(NCHUNK):
            pltpu.sync_copy(dst1_hbm.at[pl.ds((wid * NCHUNK + j) * CH, CH)],
                            idx_cur)
            if j < REAL_CHUNKS_LAST:
                pltpu.sync_copy(buf0, shared_acc.at[idx_cur], add=True)
            else:
                @pl.when(wid != NW - 1)
                def _():
                    pltpu.sync_copy(buf0, shared_acc.at[idx_cur], add=True)
        plsc.subcore_barrier()
        pltpu.sync_copy(shared_acc.at[pl.ds(sid * SROWS, SROWS)],
                        deg_hbm.at[cid].at[pl.ds(sid * SROWS, SROWS)])

    return deg_k


def _sc_deg(dst2, ones_ch, zeros_n):
    return _build_sc_deg()(dst2, ones_ch, zeros_n)


def kernel(x, edge_index, edge_attr, batch, lin0_W, lin0_b, nn1_W, nn1_b,
           nn2_W, nn2_b, conv_root, conv_bias, gru_W_ih, gru_W_hh, gru_b_ih,
           gru_b_hh, lstm_W_ih, lstm_W_hh, lstm_b_ih, lstm_b_hh, lin1_W,
           lin1_b, lin2_W, lin2_b):
    f32 = jnp.float32
    pad = E2 - E
    src2 = jnp.concatenate(
        [edge_index[0].astype(jnp.int32), jnp.zeros((pad,), jnp.int32)]
    ).reshape(NW * NCHUNK, CH)
    dst1 = jnp.concatenate(
        [edge_index[1].astype(jnp.int32), jnp.zeros((pad,), jnp.int32)])
    ea_p = jnp.zeros((E2, 8), f32).at[:E, :5].set(edge_attr)
    x_p = jnp.zeros((N, 16), f32).at[:, :11].set(x)
    batch2 = batch.astype(jnp.int32).reshape(N, 1)

    w_l0 = jnp.zeros((16, DIM), f32).at[:11].set(lin0_W.T)
    b_l0 = lin0_b.reshape(1, DIM)
    w1 = jnp.zeros((8, 128), f32).at[:5].set(nn1_W.T)
    b1 = nn1_b.reshape(1, 128)
    w2 = nn2_W.T
    b2 = nn2_b.reshape(1, DIM * DIM)
    cb = conv_bias.reshape(1, DIM)
    gwih = gru_W_ih.T
    gwhh = gru_W_hh.T
    gbih = gru_b_ih.reshape(1, 3 * DIM)
    gbhh = gru_b_hh.reshape(1, 3 * DIM)
    lwih = lstm_W_ih.T
    lwhh = lstm_W_hh.T
    lbih = lstm_b_ih.reshape(1, 4 * DIM)
    lbhh = lstm_b_hh.reshape(1, 4 * DIM)
    l1 = lin1_W.T
    l1b = lin1_b.reshape(1, DIM)
    l2 = lin2_W.T
    l2b = lin2_b.reshape(1, 1)

    bsel = jnp.repeat(jnp.eye(DIM, dtype=f32), DIM, axis=1)
    rsel = jnp.tile(jnp.eye(DIM, dtype=f32), (DIM, 1))
    zeros_n = jnp.zeros((N2, DIM), f32)
    ones_ch = jnp.ones((CH, DIM), f32)

    out = pl.pallas_call(
        _lin0_body,
        grid=(N // NBLK,),
        in_specs=[
            pl.BlockSpec((NBLK, 16), lambda i: (i, 0)),
            pl.BlockSpec((16, DIM), lambda i: (0, 0)),
            pl.BlockSpec((1, DIM), lambda i: (0, 0)),
        ],
        out_specs=pl.BlockSpec((NBLK, DIM), lambda i: (i, 0)),
        out_shape=jax.ShapeDtypeStruct((N, DIM), f32),
    )(x_p, w_l0, b_l0)

    degp = _sc_deg(dst1, ones_ch, zeros_n)

    edge_call = pl.pallas_call(
        _edge_body,
        grid=(E2 // EBLK,),
        in_specs=[
            pl.BlockSpec((EBLK, 8), lambda i: (i, 0)),
            pl.BlockSpec((EBLK, DIM), lambda i: (i, 0)),
            pl.BlockSpec((8, 128), lambda i: (0, 0)),
            pl.BlockSpec((1, 128), lambda i: (0, 0)),
            pl.BlockSpec((128, DIM * DIM), lambda i: (0, 0)),
            pl.BlockSpec((1, DIM * DIM), lambda i: (0, 0)),
            pl.BlockSpec((DIM, DIM * DIM), lambda i: (0, 0)),
            pl.BlockSpec((DIM * DIM, DIM), lambda i: (0, 0)),
        ],
        out_specs=pl.BlockSpec((EBLK, DIM), lambda i: (i, 0)),
        out_shape=jax.ShapeDtypeStruct((E2, DIM), f32),
    )

    node_call = pl.pallas_call(
        _node_body,
        grid=(N // NBLK,),
        in_specs=[
            pl.BlockSpec((NC, NBLK, DIM), lambda i: (0, i, 0)),
            pl.BlockSpec((NC, NBLK, DIM), lambda i: (0, i, 0)),
            pl.BlockSpec((NBLK, DIM), lambda i: (i, 0)),
            pl.BlockSpec((DIM, DIM), lambda i: (0, 0)),
            pl.BlockSpec((1, DIM), lambda i: (0, 0)),
            pl.BlockSpec((DIM, 3 * DIM), lambda i: (0, 0)),
            pl.BlockSpec((DIM, 3 * DIM), lambda i: (0, 0)),
            pl.BlockSpec((1, 3 * DIM), lambda i: (0, 0)),
            pl.BlockSpec((1, 3 * DIM), lambda i: (0, 0)),
        ],
        out_specs=pl.BlockSpec((NBLK, DIM), lambda i: (i, 0)),
        out_shape=jax.ShapeDtypeStruct((N, DIM), f32),
    )

    for _ in range(3):
        xj = _sc_gather(out, src2)
        msg = edge_call(ea_p, xj, w1, b1, w2, b2, bsel, rsel)
        aggp = _sc_scatter(msg, dst1, zeros_n)
        out = node_call(aggp, degp, out, conv_root, cb, gwih, gwhh, gbih, gbhh)

    z = pl.pallas_call(
        _s2s_body,
        out_shape=jax.ShapeDtypeStruct((G, 1), f32),
        scratch_shapes=[pltpu.VMEM((N, 1), f32)],
    )(out, batch2, lwih, lwhh, lbih, lbhh, l1, l1b, l2, l2b)
    return z.reshape(-1)


# speedup vs baseline: 2.6025x; 1.0011x over previous
import functools

import jax
import jax.numpy as jnp
from jax import lax
from jax.experimental import pallas as pl
from jax.experimental.pallas import tpu as pltpu
from jax.experimental.pallas import tpu_sc as plsc

DIM = 32
N = 10000
E = 160000
G = 500

NC, NS = 2, 16
NW = NC * NS
CH = 128
NCHUNK = 40
EW = CH * NCHUNK
E2 = EW * NW
REAL_CHUNKS_LAST = (E - (NW - 1) * EW) // CH

EBLK = 1024
NBLK = 2000
N2 = 10240
SROWS = N2 // NS


def _lin0_body(x_ref, w_ref, b_ref, o_ref):
    o_ref[...] = jnp.maximum(
        jnp.dot(x_ref[...], w_ref[...], preferred_element_type=jnp.float32,
                precision=lax.Precision.HIGHEST)
        + b_ref[...], 0.0)


def _edge_body(ea_ref, xj_ref, w1_ref, b1_ref, w2_ref, b2_ref, bsel_ref,
               rsel_ref, o_ref):
    hi = lax.Precision.DEFAULT
    eh = jnp.maximum(
        jnp.dot(ea_ref[...], w1_ref[...], preferred_element_type=jnp.float32,
                precision=hi)
        + b1_ref[...], 0.0)
    theta = jnp.dot(eh, w2_ref[...], preferred_element_type=jnp.float32,
                    precision=hi) + b2_ref[...]
    xj = xj_ref[...]
    x_rep = jnp.dot(xj, bsel_ref[...], preferred_element_type=jnp.float32,
                    precision=hi)
    acc = jnp.dot(theta * x_rep, rsel_ref[...],
                  preferred_element_type=jnp.float32, precision=hi)
    row = pl.program_id(0) * EBLK + lax.broadcasted_iota(jnp.int32, (EBLK, 1), 0)
    o_ref[...] = jnp.where(row < E, acc, 0.0)


def _node_body(aggp_ref, degp_ref, out_ref, cr_ref, cb_ref, wih_ref, whh_ref,
               bih_ref, bhh_ref, o_ref):
    agg = aggp_ref[0] + aggp_ref[1]
    deg = jnp.maximum(degp_ref[0] + degp_ref[1], 1.0)
    h = out_ref[...]
    m = jnp.maximum(
        agg / deg
        + jnp.dot(h, cr_ref[...], preferred_element_type=jnp.float32,
                  precision=lax.Precision.HIGHEST)
        + cb_ref[...], 0.0)
    gi = jnp.dot(m, wih_ref[...], preferred_element_type=jnp.float32,
                 precision=lax.Precision.HIGHEST) + bih_ref[...]
    gh = jnp.dot(h, whh_ref[...], preferred_element_type=jnp.float32,
                 precision=lax.Precision.HIGHEST) + bhh_ref[...]
    r = jax.nn.sigmoid(gi[:, 0:DIM] + gh[:, 0:DIM])
    z = jax.nn.sigmoid(gi[:, DIM:2 * DIM] + gh[:, DIM:2 * DIM])
    n = jnp.tanh(gi[:, 2 * DIM:] + r * gh[:, 2 * DIM:])
    o_ref[...] = (1.0 - z) * n + z * h


def _s2s_body(out_ref, b_ref, wih_ref, whh_ref, bih_ref, bhh_ref,
              l1_ref, l1b_ref, l2_ref, l2b_ref, o_ref, e_ref):
    nblk = 10
    chn = N // nblk
    gids = lax.broadcasted_iota(jnp.int32, (1, G), 1)
    dn = (((0,), (0,)), ((), ()))
    qstar = jnp.zeros((G, 2 * DIM), jnp.float32)
    hh = jnp.zeros((G, DIM), jnp.float32)
    cc = jnp.zeros((G, DIM), jnp.float32)
    for _ in range(3):
        g = (jnp.dot(qstar, wih_ref[...], preferred_element_type=jnp.float32,
                     precision=lax.Precision.HIGHEST)
             + bih_ref[...]
             + jnp.dot(hh, whh_ref[...], preferred_element_type=jnp.float32,
                       precision=lax.Precision.HIGHEST)
             + bhh_ref[...])
        ii = jax.nn.sigmoid(g[:, 0:DIM])
        ff = jax.nn.sigmoid(g[:, DIM:2 * DIM])
        gg = jnp.tanh(g[:, 2 * DIM:3 * DIM])
        oo = jax.nn.sigmoid(g[:, 3 * DIM:])
        cc = ff * cc + ii * gg
        hh = oo * jnp.tanh(cc)
        q = hh

        def pass1(bk, emax):
            st = pl.multiple_of(bk * chn, chn)
            bb = b_ref[pl.ds(st, chn), :]
            mff = (bb == gids).astype(jnp.float32)
            ob = out_ref[pl.ds(st, chn), 0:DIM]
            qg = jnp.dot(mff, q, preferred_element_type=jnp.float32,
                         precision=lax.Precision.HIGHEST)
            e = jnp.sum(ob * qg, axis=1, keepdims=True)
            e_ref[pl.ds(st, chn), :] = e
            return jnp.maximum(
                emax,
                jnp.max(mff * e + (mff - 1.0) * 1e30, axis=0, keepdims=True))

        emax = lax.fori_loop(0, nblk, pass1,
                             jnp.full((1, G), -1e30, jnp.float32))
        emax = jnp.where(emax < -1e29, 0.0, emax)

        def pass2(bk, c):
            denom, rraw = c
            st = pl.multiple_of(bk * chn, chn)
            bb = b_ref[pl.ds(st, chn), :]
            mff = (bb == gids).astype(jnp.float32)
            ob = out_ref[pl.ds(st, chn), 0:DIM]
            e = e_ref[pl.ds(st, chn), :]
            emax_n = jnp.sum(mff * emax, axis=1, keepdims=True)
            a = jnp.exp(e - emax_n)
            denom = denom + lax.dot_general(
                mff, a, dn, preferred_element_type=jnp.float32,
                precision=lax.Precision.HIGHEST)
            rraw = rraw + lax.dot_general(
                mff, a * ob, dn, preferred_element_type=jnp.float32,
                precision=lax.Precision.HIGHEST)
            return denom, rraw

        denom, rraw = lax.fori_loop(
            0, nblk, pass2,
            (jnp.zeros((G, 1), jnp.float32), jnp.zeros((G, DIM), jnp.float32)))
        r = rraw / (denom + 1e-16)
        qstar = jnp.concatenate([q, r], axis=1)
    z = jnp.maximum(
        jnp.dot(qstar, l1_ref[...], preferred_element_type=jnp.float32,
                precision=lax.Precision.HIGHEST)
        + l1b_ref[...], 0.0)
    o_ref[...] = jnp.dot(z, l2_ref[...], preferred_element_type=jnp.float32,
                         precision=lax.Precision.HIGHEST) + l2b_ref[...]


@functools.cache
def _build_sc_gather():
    mesh = plsc.VectorSubcoreMesh(core_axis_name="c", subcore_axis_name="s")

    @functools.partial(
        pl.kernel,
        mesh=mesh,
        out_type=jax.ShapeDtypeStruct((E2, DIM), jnp.float32),
        scratch_types=[
            pltpu.VMEM((NCHUNK, CH), jnp.int32),
            pltpu.VMEM((CH, DIM), jnp.float32),
            pltpu.VMEM((CH, DIM), jnp.float32),
            pltpu.VMEM((CH, DIM), jnp.float32),
            pltpu.SemaphoreType.DMA,
            pltpu.SemaphoreType.DMA,
            pltpu.SemaphoreType.DMA,
            pltpu.SemaphoreType.DMA,
            pltpu.SemaphoreType.DMA,
            pltpu.SemaphoreType.DMA,
        ],
        compiler_params=pltpu.CompilerParams(use_tc_tiling_on_sc=False),
    )
    def gather_k(nodes_hbm, src2_hbm, xj_hbm, idx_v, buf0, buf1, buf2,
                 g0, g1, g2, s0, s1, s2):
        wid = lax.axis_index("s") * NC + lax.axis_index("c")
        pltpu.sync_copy(src2_hbm.at[pl.ds(wid * NCHUNK, NCHUNK)], idx_v)
        bufs = (buf0, buf1, buf2)
        gsems = (g0, g1, g2)
        ssems = (s0, s1, s2)
        gath = {0: pltpu.async_copy(nodes_hbm.at[idx_v.at[0]], buf0, g0)}
        stores = {}
        for j in range(NCHUNK):
            if j - 2 >= 0:
                stores[j - 2].wait()
            if j + 1 < NCHUNK:
                k = (j + 1) % 3
                gath[j + 1] = pltpu.async_copy(
                    nodes_hbm.at[idx_v.at[j + 1]], bufs[k], gsems[k])
            gath[j].wait()
            stores[j] = pltpu.async_copy(
                bufs[j % 3], xj_hbm.at[pl.ds(wid * EW + j * CH, CH)],
                ssems[j % 3])
        stores[NCHUNK - 2].wait()
        stores[NCHUNK - 1].wait()

    return gather_k


def _sc_gather(nodes, src2):
    return _build_sc_gather()(nodes, src2)


@functools.cache
def _build_sc_scatter():
    mesh = plsc.VectorSubcoreMesh(core_axis_name="c", subcore_axis_name="s")

    @functools.partial(
        pl.kernel,
        mesh=mesh,
        out_type=jax.ShapeDtypeStruct((NC, N2, DIM), jnp.float32),
        scratch_types=[
            pltpu.VMEM_SHARED((N2, DIM), jnp.float32),
            pltpu.VMEM((CH,), jnp.int32),
            pltpu.VMEM((CH, DIM), jnp.float32),
            pltpu.VMEM((CH, DIM), jnp.float32),
            pltpu.SemaphoreType.DMA,
            pltpu.SemaphoreType.DMA,
        ],
        compiler_params=pltpu.CompilerParams(use_tc_tiling_on_sc=False),
    )
    def scatter_k(msg_hbm, dst1_hbm, zeros_hbm, acc_hbm, shared_acc, idx_cur,
                  buf0, buf1, sem0, sem1):
        cid = lax.axis_index("c")
        sid = lax.axis_index("s")
        wid = sid * NC + cid
        pltpu.sync_copy(zeros_hbm.at[pl.ds(sid * SROWS, SROWS)],
                        shared_acc.at[pl.ds(sid * SROWS, SROWS)])
        plsc.subcore_barrier()
        pending = pltpu.async_copy(msg_hbm.at[pl.ds(wid * EW, CH)], buf0, sem0)
        for j in range(NCHUNK):
            cbuf = buf0 if j % 2 == 0 else buf1
            nbuf, nsem = (buf1, sem1) if j % 2 == 0 else (buf0, sem0)
            nxt = None
            if j + 1 < NCHUNK:
                nxt = pltpu.async_copy(
                    msg_hbm.at[pl.ds(wid * EW + (j + 1) * CH, CH)], nbuf, nsem)
            pltpu.sync_copy(dst1_hbm.at[pl.ds((wid * NCHUNK + j) * CH, CH)],
                            idx_cur)
            pending.wait()
            pltpu.sync_copy(cbuf, shared_acc.at[idx_cur], add=True)
            pending = nxt
        plsc.subcore_barrier()
        pltpu.sync_copy(shared_acc.at[pl.ds(sid * SROWS, SROWS)],
                        acc_hbm.at[cid].at[pl.ds(sid * SROWS, SROWS)])

    return scatter_k


def _sc_scatter(msg, dst2, zeros_n):
    return _build_sc_scatter()(msg, dst2, zeros_n)


@functools.cache
def _build_sc_deg():
    mesh = plsc.VectorSubcoreMesh(core_axis_name="c", subcore_axis_name="s")

    @functools.partial(
        pl.kernel,
        mesh=mesh,
        out_type=jax.ShapeDtypeStruct((NC, N2, DIM), jnp.float32),
        scratch_types=[
            pltpu.VMEM_SHARED((N2, DIM), jnp.float32),
            pltpu.VMEM((CH,), jnp.int32),
            pltpu.VMEM((CH, DIM), jnp.float32),
        ],
        compiler_params=pltpu.CompilerParams(use_tc_tiling_on_sc=False),
    )
    def deg_k(dst1_hbm, ones_hbm, zeros_hbm, deg_hbm, shared_acc, idx_cur, buf0):
        cid = lax.axis_index("c")
        sid = lax.axis_index("s")
        wid = sid * NC + cid
        pltpu.sync_copy(zeros_hbm.at[pl.ds(sid * SROWS, SROWS)],
                        shared_acc.at[pl.ds(sid * SROWS, SROWS)])
        pltpu.sync_copy(ones_hbm, buf0)
        plsc.subcore_barrier()
        for j in range(NCHUNK):
            pltpu.sync_copy(dst1_hbm.at[pl.ds((wid * NCHUNK + j) * CH, CH)],
                            idx_cur)
            if j < REAL_CHUNKS_LAST:
                pltpu.sync_copy(buf0, shared_acc.at[idx_cur], add=True)
            else:
                @pl.when(wid != NW - 1)
                def _():
                    pltpu.sync_copy(buf0, shared_acc.at[idx_cur], add=True)
        plsc.subcore_barrier()
        pltpu.sync_copy(shared_acc.at[pl.ds(sid * SROWS, SROWS)],
                        deg_hbm.at[cid].at[pl.ds(sid * SROWS, SROWS)])

    return deg_k


def _sc_deg(dst2, ones_ch, zeros_n):
    return _build_sc_deg()(dst2, ones_ch, zeros_n)


def kernel(x, edge_index, edge_attr, batch, lin0_W, lin0_b, nn1_W, nn1_b,
           nn2_W, nn2_b, conv_root, conv_bias, gru_W_ih, gru_W_hh, gru_b_ih,
           gru_b_hh, lstm_W_ih, lstm_W_hh, lstm_b_ih, lstm_b_hh, lin1_W,
           lin1_b, lin2_W, lin2_b):
    f32 = jnp.float32
    pad = E2 - E
    src2 = jnp.concatenate(
        [edge_index[0].astype(jnp.int32), jnp.zeros((pad,), jnp.int32)]
    ).reshape(NW * NCHUNK, CH)
    dst1 = jnp.concatenate(
        [edge_index[1].astype(jnp.int32), jnp.zeros((pad,), jnp.int32)])
    ea_p = jnp.zeros((E2, 8), f32).at[:E, :5].set(edge_attr)
    x_p = jnp.zeros((N, 16), f32).at[:, :11].set(x)
    batch2 = batch.astype(jnp.int32).reshape(N, 1)

    w_l0 = jnp.zeros((16, DIM), f32).at[:11].set(lin0_W.T)
    b_l0 = lin0_b.reshape(1, DIM)
    w1 = jnp.zeros((8, 128), f32).at[:5].set(nn1_W.T)
    b1 = nn1_b.reshape(1, 128)
    w2 = nn2_W.T
    b2 = nn2_b.reshape(1, DIM * DIM)
    cb = conv_bias.reshape(1, DIM)
    gwih = gru_W_ih.T
    gwhh = gru_W_hh.T
    gbih = gru_b_ih.reshape(1, 3 * DIM)
    gbhh = gru_b_hh.reshape(1, 3 * DIM)
    lwih = lstm_W_ih.T
    lwhh = lstm_W_hh.T
    lbih = lstm_b_ih.reshape(1, 4 * DIM)
    lbhh = lstm_b_hh.reshape(1, 4 * DIM)
    l1 = lin1_W.T
    l1b = lin1_b.reshape(1, DIM)
    l2 = lin2_W.T
    l2b = lin2_b.reshape(1, 1)

    bsel = jnp.repeat(jnp.eye(DIM, dtype=f32), DIM, axis=1)
    rsel = jnp.tile(jnp.eye(DIM, dtype=f32), (DIM, 1))
    zeros_n = jnp.zeros((N2, DIM), f32)
    ones_ch = jnp.ones((CH, DIM), f32)

    out = pl.pallas_call(
        _lin0_body,
        grid=(N // NBLK,),
        in_specs=[
            pl.BlockSpec((NBLK, 16), lambda i: (i, 0)),
            pl.BlockSpec((16, DIM), lambda i: (0, 0)),
            pl.BlockSpec((1, DIM), lambda i: (0, 0)),
        ],
        out_specs=pl.BlockSpec((NBLK, DIM), lambda i: (i, 0)),
        out_shape=jax.ShapeDtypeStruct((N, DIM), f32),
    )(x_p, w_l0, b_l0)

    degp = _sc_deg(dst1, ones_ch, zeros_n)

    edge_call = pl.pallas_call(
        _edge_body,
        grid=(E2 // EBLK,),
        in_specs=[
            pl.BlockSpec((EBLK, 8), lambda i: (i, 0)),
            pl.BlockSpec((EBLK, DIM), lambda i: (i, 0)),
            pl.BlockSpec((8, 128), lambda i: (0, 0)),
            pl.BlockSpec((1, 128), lambda i: (0, 0)),
            pl.BlockSpec((128, DIM * DIM), lambda i: (0, 0)),
            pl.BlockSpec((1, DIM * DIM), lambda i: (0, 0)),
            pl.BlockSpec((DIM, DIM * DIM), lambda i: (0, 0)),
            pl.BlockSpec((DIM * DIM, DIM), lambda i: (0, 0)),
        ],
        out_specs=pl.BlockSpec((EBLK, DIM), lambda i: (i, 0)),
        out_shape=jax.ShapeDtypeStruct((E2, DIM), f32),
    )

    node_call = pl.pallas_call(
        _node_body,
        grid=(N // NBLK,),
        in_specs=[
            pl.BlockSpec((NC, NBLK, DIM), lambda i: (0, i, 0)),
            pl.BlockSpec((NC, NBLK, DIM), lambda i: (0, i, 0)),
            pl.BlockSpec((NBLK, DIM), lambda i: (i, 0)),
            pl.BlockSpec((DIM, DIM), lambda i: (0, 0)),
            pl.BlockSpec((1, DIM), lambda i: (0, 0)),
            pl.BlockSpec((DIM, 3 * DIM), lambda i: (0, 0)),
            pl.BlockSpec((DIM, 3 * DIM), lambda i: (0, 0)),
            pl.BlockSpec((1, 3 * DIM), lambda i: (0, 0)),
            pl.BlockSpec((1, 3 * DIM), lambda i: (0, 0)),
        ],
        out_specs=pl.BlockSpec((NBLK, DIM), lambda i: (i, 0)),
        out_shape=jax.ShapeDtypeStruct((N, DIM), f32),
    )

    for _ in range(3):
        xj = _sc_gather(out, src2)
        msg = edge_call(ea_p, xj, w1, b1, w2, b2, bsel, rsel)
        aggp = _sc_scatter(msg, dst1, zeros_n)
        out = node_call(aggp, degp, out, conv_root, cb, gwih, gwhh, gbih, gbhh)

    z = pl.pallas_call(
        _s2s_body,
        out_shape=jax.ShapeDtypeStruct((G, 1), f32),
        scratch_shapes=[pltpu.VMEM((N, 1), f32)],
    )(out, batch2, lwih, lwhh, lbih, lbhh, l1, l1b, l2, l2b)
    return z.reshape(-1)


# speedup vs baseline: 2.6937x; 1.0351x over previous
import functools

import jax
import jax.numpy as jnp
from jax import lax
from jax.experimental import pallas as pl
from jax.experimental.pallas import tpu as pltpu
from jax.experimental.pallas import tpu_sc as plsc

DIM = 32
N = 10000
E = 160000
G = 500

NC, NS = 2, 16
NW = NC * NS
CH = 128
NCHUNK = 40
EW = CH * NCHUNK
E2 = EW * NW
REAL_CHUNKS_LAST = (E - (NW - 1) * EW) // CH

EBLK = 2048
NBLK = 2000
N2 = 10240
SROWS = N2 // NS


def _lin0_body(x_ref, w_ref, b_ref, o_ref):
    o_ref[...] = jnp.maximum(
        jnp.dot(x_ref[...], w_ref[...], preferred_element_type=jnp.float32,
                precision=lax.Precision.HIGHEST)
        + b_ref[...], 0.0)


def _edge_body(ea_ref, xj_ref, w1_ref, b1_ref, w2_ref, b2_ref, bsel_ref,
               rsel_ref, o_ref):
    hi = lax.Precision.DEFAULT
    eh = jnp.maximum(
        jnp.dot(ea_ref[...], w1_ref[...], preferred_element_type=jnp.float32,
                precision=hi)
        + b1_ref[...], 0.0)
    theta = jnp.dot(eh, w2_ref[...], preferred_element_type=jnp.float32,
                    precision=hi) + b2_ref[...]
    xj = xj_ref[...]
    x_rep = jnp.dot(xj, bsel_ref[...], preferred_element_type=jnp.float32,
                    precision=hi)
    acc = jnp.dot(theta * x_rep, rsel_ref[...],
                  preferred_element_type=jnp.float32, precision=hi)
    row = pl.program_id(0) * EBLK + lax.broadcasted_iota(jnp.int32, (EBLK, 1), 0)
    o_ref[...] = jnp.where(row < E, acc, 0.0)


def _node_body(aggp_ref, degp_ref, out_ref, cr_ref, cb_ref, wih_ref, whh_ref,
               bih_ref, bhh_ref, o_ref):
    agg = aggp_ref[0] + aggp_ref[1]
    deg = jnp.maximum(degp_ref[0] + degp_ref[1], 1.0)
    h = out_ref[...]
    m = jnp.maximum(
        agg / deg
        + jnp.dot(h, cr_ref[...], preferred_element_type=jnp.float32,
                  precision=lax.Precision.HIGHEST)
        + cb_ref[...], 0.0)
    gi = jnp.dot(m, wih_ref[...], preferred_element_type=jnp.float32,
                 precision=lax.Precision.HIGHEST) + bih_ref[...]
    gh = jnp.dot(h, whh_ref[...], preferred_element_type=jnp.float32,
                 precision=lax.Precision.HIGHEST) + bhh_ref[...]
    r = jax.nn.sigmoid(gi[:, 0:DIM] + gh[:, 0:DIM])
    z = jax.nn.sigmoid(gi[:, DIM:2 * DIM] + gh[:, DIM:2 * DIM])
    n = jnp.tanh(gi[:, 2 * DIM:] + r * gh[:, 2 * DIM:])
    o_ref[...] = (1.0 - z) * n + z * h


def _s2s_body(out_ref, b_ref, wih_ref, whh_ref, bih_ref, bhh_ref,
              l1_ref, l1b_ref, l2_ref, l2b_ref, o_ref, e_ref):
    nblk = 10
    chn = N // nblk
    gids = lax.broadcasted_iota(jnp.int32, (1, G), 1)
    dn = (((0,), (0,)), ((), ()))
    qstar = jnp.zeros((G, 2 * DIM), jnp.float32)
    hh = jnp.zeros((G, DIM), jnp.float32)
    cc = jnp.zeros((G, DIM), jnp.float32)
    for _ in range(3):
        g = (jnp.dot(qstar, wih_ref[...], preferred_element_type=jnp.float32,
                     precision=lax.Precision.HIGHEST)
             + bih_ref[...]
             + jnp.dot(hh, whh_ref[...], preferred_element_type=jnp.float32,
                       precision=lax.Precision.HIGHEST)
             + bhh_ref[...])
        ii = jax.nn.sigmoid(g[:, 0:DIM])
        ff = jax.nn.sigmoid(g[:, DIM:2 * DIM])
        gg = jnp.tanh(g[:, 2 * DIM:3 * DIM])
        oo = jax.nn.sigmoid(g[:, 3 * DIM:])
        cc = ff * cc + ii * gg
        hh = oo * jnp.tanh(cc)
        q = hh

        def pass1(bk, emax):
            st = pl.multiple_of(bk * chn, chn)
            bb = b_ref[pl.ds(st, chn), :]
            mff = (bb == gids).astype(jnp.float32)
            ob = out_ref[pl.ds(st, chn), 0:DIM]
            qg = jnp.dot(mff, q, preferred_element_type=jnp.float32,
                         precision=lax.Precision.HIGHEST)
            e = jnp.sum(ob * qg, axis=1, keepdims=True)
            e_ref[pl.ds(st, chn), :] = e
            return jnp.maximum(
                emax,
                jnp.max(mff * e + (mff - 1.0) * 1e30, axis=0, keepdims=True))

        emax = lax.fori_loop(0, nblk, pass1,
                             jnp.full((1, G), -1e30, jnp.float32))
        emax = jnp.where(emax < -1e29, 0.0, emax)

        def pass2(bk, c):
            denom, rraw = c
            st = pl.multiple_of(bk * chn, chn)
            bb = b_ref[pl.ds(st, chn), :]
            mff = (bb == gids).astype(jnp.float32)
            ob = out_ref[pl.ds(st, chn), 0:DIM]
            e = e_ref[pl.ds(st, chn), :]
            emax_n = jnp.sum(mff * emax, axis=1, keepdims=True)
            a = jnp.exp(e - emax_n)
            denom = denom + lax.dot_general(
                mff, a, dn, preferred_element_type=jnp.float32,
                precision=lax.Precision.HIGHEST)
            rraw = rraw + lax.dot_general(
                mff, a * ob, dn, preferred_element_type=jnp.float32,
                precision=lax.Precision.HIGHEST)
            return denom, rraw

        denom, rraw = lax.fori_loop(
            0, nblk, pass2,
            (jnp.zeros((G, 1), jnp.float32), jnp.zeros((G, DIM), jnp.float32)))
        r = rraw / (denom + 1e-16)
        qstar = jnp.concatenate([q, r], axis=1)
    z = jnp.maximum(
        jnp.dot(qstar, l1_ref[...], preferred_element_type=jnp.float32,
                precision=lax.Precision.HIGHEST)
        + l1b_ref[...], 0.0)
    o_ref[...] = jnp.dot(z, l2_ref[...], preferred_element_type=jnp.float32,
                         precision=lax.Precision.HIGHEST) + l2b_ref[...]


@functools.cache
def _build_sc_gather():
    mesh = plsc.VectorSubcoreMesh(core_axis_name="c", subcore_axis_name="s")

    @functools.partial(
        pl.kernel,
        mesh=mesh,
        out_type=jax.ShapeDtypeStruct((E2, DIM), jnp.float32),
        scratch_types=[
            pltpu.VMEM((NCHUNK, CH), jnp.int32),
            pltpu.VMEM((CH, DIM), jnp.float32),
            pltpu.VMEM((CH, DIM), jnp.float32),
            pltpu.VMEM((CH, DIM), jnp.float32),
            pltpu.SemaphoreType.DMA,
            pltpu.SemaphoreType.DMA,
            pltpu.SemaphoreType.DMA,
            pltpu.SemaphoreType.DMA,
            pltpu.SemaphoreType.DMA,
            pltpu.SemaphoreType.DMA,
        ],
        compiler_params=pltpu.CompilerParams(use_tc_tiling_on_sc=False),
    )
    def gather_k(nodes_hbm, src2_hbm, xj_hbm, idx_v, buf0, buf1, buf2,
                 g0, g1, g2, s0, s1, s2):
        wid = lax.axis_index("s") * NC + lax.axis_index("c")
        pltpu.sync_copy(src2_hbm.at[pl.ds(wid * NCHUNK, NCHUNK)], idx_v)
        bufs = (buf0, buf1, buf2)
        gsems = (g0, g1, g2)
        ssems = (s0, s1, s2)
        gath = {0: pltpu.async_copy(nodes_hbm.at[idx_v.at[0]], buf0, g0)}
        stores = {}
        for j in range(NCHUNK):
            if j - 2 >= 0:
                stores[j - 2].wait()
            if j + 1 < NCHUNK:
                k = (j + 1) % 3
                gath[j + 1] = pltpu.async_copy(
                    nodes_hbm.at[idx_v.at[j + 1]], bufs[k], gsems[k])
            gath[j].wait()
            stores[j] = pltpu.async_copy(
                bufs[j % 3], xj_hbm.at[pl.ds(wid * EW + j * CH, CH)],
                ssems[j % 3])
        stores[NCHUNK - 2].wait()
        stores[NCHUNK - 1].wait()

    return gather_k


def _sc_gather(nodes, src2):
    return _build_sc_gather()(nodes, src2)


@functools.cache
def _build_sc_scatter():
    mesh = plsc.VectorSubcoreMesh(core_axis_name="c", subcore_axis_name="s")

    @functools.partial(
        pl.kernel,
        mesh=mesh,
        out_type=jax.ShapeDtypeStruct((NC, N2, DIM), jnp.float32),
        scratch_types=[
            pltpu.VMEM_SHARED((N2, DIM), jnp.float32),
            pltpu.VMEM((CH,), jnp.int32),
            pltpu.VMEM((CH, DIM), jnp.float32),
            pltpu.VMEM((CH, DIM), jnp.float32),
            pltpu.SemaphoreType.DMA,
            pltpu.SemaphoreType.DMA,
        ],
        compiler_params=pltpu.CompilerParams(use_tc_tiling_on_sc=False),
    )
    def scatter_k(msg_hbm, dst1_hbm, zeros_hbm, acc_hbm, shared_acc, idx_cur,
                  buf0, buf1, sem0, sem1):
        cid = lax.axis_index("c")
        sid = lax.axis_index("s")
        wid = sid * NC + cid
        pltpu.sync_copy(zeros_hbm.at[pl.ds(sid * SROWS, SROWS)],
                        shared_acc.at[pl.ds(sid * SROWS, SROWS)])
        plsc.subcore_barrier()
        pending = pltpu.async_copy(msg_hbm.at[pl.ds(wid * EW, CH)], buf0, sem0)
        for j in range(NCHUNK):
            cbuf = buf0 if j % 2 == 0 else buf1
            nbuf, nsem = (buf1, sem1) if j % 2 == 0 else (buf0, sem0)
            nxt = None
            if j + 1 < NCHUNK:
                nxt = pltpu.async_copy(
                    msg_hbm.at[pl.ds(wid * EW + (j + 1) * CH, CH)], nbuf, nsem)
            pltpu.sync_copy(dst1_hbm.at[pl.ds((wid * NCHUNK + j) * CH, CH)],
                            idx_cur)
            pending.wait()
            pltpu.sync_copy(cbuf, shared_acc.at[idx_cur], add=True)
            pending = nxt
        plsc.subcore_barrier()
        pltpu.sync_copy(shared_acc.at[pl.ds(sid * SROWS, SROWS)],
                        acc_hbm.at[cid].at[pl.ds(sid * SROWS, SROWS)])

    return scatter_k


def _sc_scatter(msg, dst2, zeros_n):
    return _build_sc_scatter()(msg, dst2, zeros_n)


@functools.cache
def _build_sc_deg():
    mesh = plsc.VectorSubcoreMesh(core_axis_name="c", subcore_axis_name="s")

    @functools.partial(
        pl.kernel,
        mesh=mesh,
        out_type=jax.ShapeDtypeStruct((NC, N2, DIM), jnp.float32),
        scratch_types=[
            pltpu.VMEM_SHARED((N2, DIM), jnp.float32),
            pltpu.VMEM((CH,), jnp.int32),
            pltpu.VMEM((CH, DIM), jnp.float32),
        ],
        compiler_params=pltpu.CompilerParams(use_tc_tiling_on_sc=False),
    )
    def deg_k(dst1_hbm, ones_hbm, zeros_hbm, deg_hbm, shared_acc, idx_cur, buf0):
        cid = lax.axis_index("c")
        sid = lax.axis_index("s")
        wid = sid * NC + cid
        pltpu.sync_copy(zeros_hbm.at[pl.ds(sid * SROWS, SROWS)],
                        shared_acc.at[pl.ds(sid * SROWS, SROWS)])
        pltpu.sync_copy(ones_hbm, buf0)
        plsc.subcore_barrier()
        for j in range(NCHUNK):
            pltpu.sync_copy(dst1_hbm.at[pl.ds((wid * NCHUNK + j) * CH, CH)],
                            idx_cur)
            if j < REAL_CHUNKS_LAST:
                pltpu.sync_copy(buf0, shared_acc.at[idx_cur], add=True)
            else:
                @pl.when(wid != NW - 1)
                def _():
                    pltpu.sync_copy(buf0, shared_acc.at[idx_cur], add=True)
        plsc.subcore_barrier()
        pltpu.sync_copy(shared_acc.at[pl.ds(sid * SROWS, SROWS)],
                        deg_hbm.at[cid].at[pl.ds(sid * SROWS, SROWS)])

    return deg_k


def _sc_deg(dst2, ones_ch, zeros_n):
    return _build_sc_deg()(dst2, ones_ch, zeros_n)


def kernel(x, edge_index, edge_attr, batch, lin0_W, lin0_b, nn1_W, nn1_b,
           nn2_W, nn2_b, conv_root, conv_bias, gru_W_ih, gru_W_hh, gru_b_ih,
           gru_b_hh, lstm_W_ih, lstm_W_hh, lstm_b_ih, lstm_b_hh, lin1_W,
           lin1_b, lin2_W, lin2_b):
    f32 = jnp.float32
    pad = E2 - E
    src2 = jnp.concatenate(
        [edge_index[0].astype(jnp.int32), jnp.zeros((pad,), jnp.int32)]
    ).reshape(NW * NCHUNK, CH)
    dst1 = jnp.concatenate(
        [edge_index[1].astype(jnp.int32), jnp.zeros((pad,), jnp.int32)])
    ea_p = jnp.zeros((E2, 8), f32).at[:E, :5].set(edge_attr)
    x_p = jnp.zeros((N, 16), f32).at[:, :11].set(x)
    batch2 = batch.astype(jnp.int32).reshape(N, 1)

    w_l0 = jnp.zeros((16, DIM), f32).at[:11].set(lin0_W.T)
    b_l0 = lin0_b.reshape(1, DIM)
    w1 = jnp.zeros((8, 128), f32).at[:5].set(nn1_W.T)
    b1 = nn1_b.reshape(1, 128)
    w2 = nn2_W.T
    b2 = nn2_b.reshape(1, DIM * DIM)
    cb = conv_bias.reshape(1, DIM)
    gwih = gru_W_ih.T
    gwhh = gru_W_hh.T
    gbih = gru_b_ih.reshape(1, 3 * DIM)
    gbhh = gru_b_hh.reshape(1, 3 * DIM)
    lwih = lstm_W_ih.T
    lwhh = lstm_W_hh.T
    lbih = lstm_b_ih.reshape(1, 4 * DIM)
    lbhh = lstm_b_hh.reshape(1, 4 * DIM)
    l1 = lin1_W.T
    l1b = lin1_b.reshape(1, DIM)
    l2 = lin2_W.T
    l2b = lin2_b.reshape(1, 1)

    bsel = jnp.repeat(jnp.eye(DIM, dtype=f32), DIM, axis=1)
    rsel = jnp.tile(jnp.eye(DIM, dtype=f32), (DIM, 1))
    zeros_n = jnp.zeros((N2, DIM), f32)
    ones_ch = jnp.ones((CH, DIM), f32)

    out = pl.pallas_call(
        _lin0_body,
        grid=(N // NBLK,),
        in_specs=[
            pl.BlockSpec((NBLK, 16), lambda i: (i, 0)),
            pl.BlockSpec((16, DIM), lambda i: (0, 0)),
            pl.BlockSpec((1, DIM), lambda i: (0, 0)),
        ],
        out_specs=pl.BlockSpec((NBLK, DIM), lambda i: (i, 0)),
        out_shape=jax.ShapeDtypeStruct((N, DIM), f32),
    )(x_p, w_l0, b_l0)

    degp = _sc_deg(dst1, ones_ch, zeros_n)

    edge_call = pl.pallas_call(
        _edge_body,
        grid=(E2 // EBLK,),
        in_specs=[
            pl.BlockSpec((EBLK, 8), lambda i: (i, 0)),
            pl.BlockSpec((EBLK, DIM), lambda i: (i, 0)),
            pl.BlockSpec((8, 128), lambda i: (0, 0)),
            pl.BlockSpec((1, 128), lambda i: (0, 0)),
            pl.BlockSpec((128, DIM * DIM), lambda i: (0, 0)),
            pl.BlockSpec((1, DIM * DIM), lambda i: (0, 0)),
            pl.BlockSpec((DIM, DIM * DIM), lambda i: (0, 0)),
            pl.BlockSpec((DIM * DIM, DIM), lambda i: (0, 0)),
        ],
        out_specs=pl.BlockSpec((EBLK, DIM), lambda i: (i, 0)),
        out_shape=jax.ShapeDtypeStruct((E2, DIM), f32),
    )

    node_call = pl.pallas_call(
        _node_body,
        grid=(N // NBLK,),
        in_specs=[
            pl.BlockSpec((NC, NBLK, DIM), lambda i: (0, i, 0)),
            pl.BlockSpec((NC, NBLK, DIM), lambda i: (0, i, 0)),
            pl.BlockSpec((NBLK, DIM), lambda i: (i, 0)),
            pl.BlockSpec((DIM, DIM), lambda i: (0, 0)),
            pl.BlockSpec((1, DIM), lambda i: (0, 0)),
            pl.BlockSpec((DIM, 3 * DIM), lambda i: (0, 0)),
            pl.BlockSpec((DIM, 3 * DIM), lambda i: (0, 0)),
            pl.BlockSpec((1, 3 * DIM), lambda i: (0, 0)),
            pl.BlockSpec((1, 3 * DIM), lambda i: (0, 0)),
        ],
        out_specs=pl.BlockSpec((NBLK, DIM), lambda i: (i, 0)),
        out_shape=jax.ShapeDtypeStruct((N, DIM), f32),
    )

    for _ in range(3):
        xj = _sc_gather(out, src2)
        msg = edge_call(ea_p, xj, w1, b1, w2, b2, bsel, rsel)
        aggp = _sc_scatter(msg, dst1, zeros_n)
        out = node_call(aggp, degp, out, conv_root, cb, gwih, gwhh, gbih, gbhh)

    z = pl.pallas_call(
        _s2s_body,
        out_shape=jax.ShapeDtypeStruct((G, 1), f32),
        scratch_shapes=[pltpu.VMEM((N, 1), f32)],
    )(out, batch2, lwih, lwhh, lbih, lbhh, l1, l1b, l2, l2b)
    return z.reshape(-1)
